# baseline (speedup 1.0000x reference)
import jax
import jax.numpy as jnp
from jax import lax
from jax.experimental import pallas as pl
from jax.experimental.pallas import tpu as pltpu

N = 32768
D = 2048
K = 512
RB = 1024
NB = N // RB
CAP = 128
M = NB * CAP
CH = 512


def _transpose_planes(planes, eye):
    return [lax.dot_general(pln, eye, (((0,), (0,)), ((), ())),
                            preferred_element_type=jnp.float32)
            for pln in planes]


def _score_select_kernel(x_ref, wt_ref, b_ref, idx_ref, p_ref, pf_ref, if_ref):
    g = pl.program_id(0)
    lt = lax.dot_general(wt_ref[...], x_ref[...], (((0,), (1,)), ((), ())),
                         preferred_element_type=jnp.float32) + b_ref[...]
    m = jnp.max(lt, axis=0, keepdims=True)
    e = jnp.exp(lt - m)
    p_ref[g] = e[1:2, :] / (e[0:1, :] + e[1:2, :])

    @pl.when(g == NB - 1)
    def _select():
        u3 = lax.bitcast_convert_type(p_ref[...], jnp.int32)

        def bit_body(t, pfx):
            cand = pfx | (jnp.int32(1) << (jnp.int32(30) - t))
            c = jnp.sum(jnp.where(u3 >= cand, jnp.int32(1), jnp.int32(0)))
            return jnp.where(c >= K, cand, pfx)
        vstar = lax.fori_loop(0, 31, bit_body, jnp.int32(0))

        utri = (lax.broadcasted_iota(jnp.int32, (RB, RB), 0)
                <= lax.broadcasted_iota(jnp.int32, (RB, RB), 1)
                ).astype(jnp.float32)
        c_col = lax.broadcasted_iota(jnp.int32, (CAP, 1), 0).astype(jnp.float32)
        j_row = lax.broadcasted_iota(jnp.int32, (1, RB), 1).astype(jnp.float32)

        def compact_body(b, carry):
            prow = p_ref[b]
            mrow = (lax.bitcast_convert_type(prow, jnp.int32)
                    >= vstar).astype(jnp.float32)
            cs = jnp.dot(mrow, utri,
                         preferred_element_type=jnp.float32)
            destv = jnp.where(mrow > 0, cs - mrow, -1.0)
            comp = (destv == c_col).astype(jnp.float32)
            fil = jnp.sum(comp, axis=1, keepdims=True)
            pf = jnp.sum(comp * prow, axis=1, keepdims=True)
            jf = jnp.sum(comp * j_row, axis=1, keepdims=True)
            pf_ref[pl.ds(b * CAP, CAP), :] = pf + fil - 1.0
            if_ref[pl.ds(b * CAP, CAP), :] = (
                jf + (b * RB).astype(jnp.float32) * fil)
            return carry
        lax.fori_loop(0, NB, compact_body, jnp.int32(0))

        p_col = pf_ref[...]
        i_col = if_ref[...]
        u_col = lax.bitcast_convert_type(p_col, jnp.int32)
        i_int = i_col.astype(jnp.int32)

        eye = (lax.broadcasted_iota(jnp.int32, (CH, CH), 0)
               == lax.broadcasted_iota(jnp.int32, (CH, CH), 1)
               ).astype(jnp.float32)
        p_rows, i_rows = [], []
        for t in range(M // CH):
            uc = u_col[t * CH:(t + 1) * CH, :]
            ic = i_int[t * CH:(t + 1) * CH, :]
            ub = [((uc >> s) & 255).astype(jnp.float32) for s in (0, 8, 16, 24)]
            ib = [(ic & 255).astype(jnp.float32),
                  (ic >> 8).astype(jnp.float32)]
            ubr = _transpose_planes(ub, eye)
            ibr = _transpose_planes(ib, eye)
            ur = (ubr[0].astype(jnp.int32)
                  | (ubr[1].astype(jnp.int32) << 8)
                  | (ubr[2].astype(jnp.int32) << 16)
                  | (ubr[3].astype(jnp.int32) << 24))
            p_rows.append(lax.bitcast_convert_type(ur, jnp.float32))
            i_rows.append(ibr[0] + ibr[1] * 256.0)
        p_row = jnp.concatenate(p_rows, axis=1)
        i_row = jnp.concatenate(i_rows, axis=1)

        r_row = lax.broadcasted_iota(jnp.int32, (1, K), 1).astype(jnp.float32)
        acc_hi = jnp.zeros((1, K), jnp.float32)
        acc_lo = jnp.zeros((1, K), jnp.float32)
        for t in range(M // CH):
            pd = p_col[t * CH:(t + 1) * CH, :]
            idd = i_col[t * CH:(t + 1) * CH, :]
            ic = i_int[t * CH:(t + 1) * CH, :]
            cmp = (p_row > pd) | ((p_row == pd) & (i_row < idd))
            rank = jnp.sum(cmp.astype(jnp.float32), axis=1, keepdims=True)
            onehot = (rank == r_row).astype(jnp.float32)
            hi = (ic >> 8).astype(jnp.float32)
            lo = (ic & 255).astype(jnp.float32)
            acc_hi = acc_hi + lax.dot_general(
                hi, onehot, (((0,), (0,)), ((), ())),
                preferred_element_type=jnp.float32)
            acc_lo = acc_lo + lax.dot_general(
                lo, onehot, (((0,), (0,)), ((), ())),
                preferred_element_type=jnp.float32)
        idx_ref[...] = (acc_hi * 256.0 + acc_lo).astype(jnp.int32)


def _gather_kernel(idx_ref, x_ref, o_ref):
    o_ref[...] = x_ref[...]


@jax.jit
def kernel(x, W, b):
    top_idx = pl.pallas_call(
        _score_select_kernel,
        grid=(NB,),
        in_specs=[
            pl.BlockSpec((RB, D), lambda g: (g, 0)),
            pl.BlockSpec((D, 2), lambda g: (0, 0)),
            pl.BlockSpec((2, 1), lambda g: (0, 0)),
        ],
        out_specs=pl.BlockSpec((1, K), lambda g: (0, 0)),
        out_shape=jax.ShapeDtypeStruct((1, K), jnp.int32),
        scratch_shapes=[
            pltpu.VMEM((NB, 1, RB), jnp.float32),
            pltpu.VMEM((M, 1), jnp.float32),
            pltpu.VMEM((M, 1), jnp.float32),
        ],
    )(x, W.T, b.reshape(2, 1))

    out = pl.pallas_call(
        _gather_kernel,
        grid_spec=pltpu.PrefetchScalarGridSpec(
            num_scalar_prefetch=1,
            grid=(K,),
            in_specs=[pl.BlockSpec((1, 1, D), lambda i, idx: (idx[i], 0, 0))],
            out_specs=pl.BlockSpec((1, 1, D), lambda i, idx: (i, 0, 0)),
        ),
        out_shape=jax.ShapeDtypeStruct((K, 1, D), jnp.float32),
    )(top_idx.reshape(K), x.reshape(N, 1, D))
    return out.reshape(K, D)


# speedup vs baseline: 2.3463x; 2.3463x over previous
import jax
import jax.numpy as jnp
from jax import lax
from jax.experimental import pallas as pl
from jax.experimental.pallas import tpu as pltpu

N = 32768
D = 2048
K = 512
RB = 1024
NB = N // RB
CAP = 64
M = NB * CAP
CH = 512
DWIN = 16


def _transpose_planes(planes, eye):
    return [lax.dot_general(pln, eye, (((0,), (0,)), ((), ())),
                            preferred_element_type=jnp.float32)
            for pln in planes]


def _score_select_kernel(x_ref, wt_ref, b_ref, idx_ref, p_ref, pf_ref, if_ref):
    g = pl.program_id(0)
    lt = lax.dot_general(wt_ref[...], x_ref[...], (((0,), (1,)), ((), ())),
                         preferred_element_type=jnp.float32) + b_ref[...]
    m = jnp.max(lt, axis=0, keepdims=True)
    e = jnp.exp(lt - m)
    p_ref[g] = e[1:2, :] / (e[0:1, :] + e[1:2, :])

    @pl.when(g == NB - 1)
    def _select():
        u3 = lax.bitcast_convert_type(p_ref[...], jnp.int32)

        def bit_body(t, pfx):
            cand = pfx | (jnp.int32(1) << (jnp.int32(30) - t))
            c = jnp.sum(jnp.where(u3 >= cand, jnp.int32(1), jnp.int32(0)))
            return jnp.where(c >= K, cand, pfx)
        vstar = lax.fori_loop(0, 31, bit_body, jnp.int32(0))

        utri = (lax.broadcasted_iota(jnp.int32, (RB, RB), 0)
                <= lax.broadcasted_iota(jnp.int32, (RB, RB), 1)
                ).astype(jnp.float32)
        c_col = lax.broadcasted_iota(jnp.int32, (CAP, 1), 0).astype(jnp.float32)
        j_row = lax.broadcasted_iota(jnp.int32, (1, RB), 1).astype(jnp.float32)

        def compact_body(b, carry):
            prow = p_ref[b]
            mrow = (lax.bitcast_convert_type(prow, jnp.int32)
                    >= vstar).astype(jnp.float32)
            cs = jnp.dot(mrow, utri,
                         preferred_element_type=jnp.float32)
            destv = jnp.where(mrow > 0, cs - mrow, -1.0)
            comp = (destv == c_col).astype(jnp.float32)
            fil = jnp.sum(comp, axis=1, keepdims=True)
            pf = jnp.sum(comp * prow, axis=1, keepdims=True)
            jf = jnp.sum(comp * j_row, axis=1, keepdims=True)
            pf_ref[pl.ds(b * CAP, CAP), :] = pf + fil - 1.0
            if_ref[pl.ds(b * CAP, CAP), :] = (
                jf + (b * RB).astype(jnp.float32) * fil)
            return carry
        lax.fori_loop(0, NB, compact_body, jnp.int32(0))

        p_col = pf_ref[...]
        i_col = if_ref[...]
        u_col = lax.bitcast_convert_type(p_col, jnp.int32)
        i_int = i_col.astype(jnp.int32)

        eye = (lax.broadcasted_iota(jnp.int32, (CH, CH), 0)
               == lax.broadcasted_iota(jnp.int32, (CH, CH), 1)
               ).astype(jnp.float32)
        p_rows, i_rows = [], []
        for t in range(M // CH):
            uc = u_col[t * CH:(t + 1) * CH, :]
            ic = i_int[t * CH:(t + 1) * CH, :]
            ub = [((uc >> s) & 255).astype(jnp.float32) for s in (0, 8, 16, 24)]
            ib = [(ic & 255).astype(jnp.float32),
                  (ic >> 8).astype(jnp.float32)]
            ubr = _transpose_planes(ub, eye)
            ibr = _transpose_planes(ib, eye)
            ur = (ubr[0].astype(jnp.int32)
                  | (ubr[1].astype(jnp.int32) << 8)
                  | (ubr[2].astype(jnp.int32) << 16)
                  | (ubr[3].astype(jnp.int32) << 24))
            p_rows.append(lax.bitcast_convert_type(ur, jnp.float32))
            i_rows.append(ibr[0] + ibr[1] * 256.0)
        p_row = jnp.concatenate(p_rows, axis=1)
        i_row = jnp.concatenate(i_rows, axis=1)

        r_row = lax.broadcasted_iota(jnp.int32, (1, K), 1).astype(jnp.float32)
        acc_hi = jnp.zeros((1, K), jnp.float32)
        acc_lo = jnp.zeros((1, K), jnp.float32)
        for t in range(M // CH):
            pd = p_col[t * CH:(t + 1) * CH, :]
            idd = i_col[t * CH:(t + 1) * CH, :]
            ic = i_int[t * CH:(t + 1) * CH, :]
            cmp = (p_row > pd) | ((p_row == pd) & (i_row < idd))
            rank = jnp.sum(cmp.astype(jnp.float32), axis=1, keepdims=True)
            onehot = (rank == r_row).astype(jnp.float32)
            hi = (ic >> 8).astype(jnp.float32)
            lo = (ic & 255).astype(jnp.float32)
            acc_hi = acc_hi + lax.dot_general(
                hi, onehot, (((0,), (0,)), ((), ())),
                preferred_element_type=jnp.float32)
            acc_lo = acc_lo + lax.dot_general(
                lo, onehot, (((0,), (0,)), ((), ())),
                preferred_element_type=jnp.float32)
        idx_ref[...] = (acc_hi * 256.0 + acc_lo).astype(jnp.int32)


def _gather_kernel(idx_ref, x_ref, o_ref, sem):
    def start(j):
        pltpu.make_async_copy(x_ref.at[pl.ds(idx_ref[j], 1)],
                              o_ref.at[pl.ds(j, 1)],
                              sem.at[j % DWIN]).start()

    for j in range(DWIN):
        start(j)

    def body(i, carry):
        pltpu.make_async_copy(x_ref.at[pl.ds(idx_ref[i], 1)],
                              o_ref.at[pl.ds(i, 1)],
                              sem.at[i % DWIN]).wait()

        @pl.when(i + DWIN < K)
        def _():
            start(i + DWIN)
        return carry
    lax.fori_loop(0, K, body, jnp.int32(0))


@jax.jit
def kernel(x, W, b):
    top_idx = pl.pallas_call(
        _score_select_kernel,
        grid=(NB,),
        in_specs=[
            pl.BlockSpec((RB, D), lambda g: (g, 0)),
            pl.BlockSpec((D, 2), lambda g: (0, 0)),
            pl.BlockSpec((2, 1), lambda g: (0, 0)),
        ],
        out_specs=pl.BlockSpec((1, K), lambda g: (0, 0)),
        out_shape=jax.ShapeDtypeStruct((1, K), jnp.int32),
        scratch_shapes=[
            pltpu.VMEM((NB, 1, RB), jnp.float32),
            pltpu.VMEM((M, 1), jnp.float32),
            pltpu.VMEM((M, 1), jnp.float32),
        ],
    )(x, W.T, b.reshape(2, 1))

    out = pl.pallas_call(
        _gather_kernel,
        grid_spec=pltpu.PrefetchScalarGridSpec(
            num_scalar_prefetch=1,
            grid=(1,),
            in_specs=[pl.BlockSpec(memory_space=pl.ANY)],
            out_specs=pl.BlockSpec(memory_space=pl.ANY),
            scratch_shapes=[pltpu.SemaphoreType.DMA((DWIN,))],
        ),
        out_shape=jax.ShapeDtypeStruct((K, D), jnp.float32),
    )(top_idx.reshape(K), x)
    return out


# speedup vs baseline: 4.2504x; 1.8115x over previous
import jax
import jax.numpy as jnp
from jax import lax
from jax.experimental import pallas as pl
from jax.experimental.pallas import tpu as pltpu

N = 32768
D = 2048
K = 512
RB = 1024
NB = N // RB
CAP = 64
M = NB * CAP
CH = 512
DWIN = 16


def _transpose_planes(planes, eye):
    return [lax.dot_general(pln, eye, (((0,), (0,)), ((), ())),
                            preferred_element_type=jnp.float32)
            for pln in planes]


def _score_select_kernel(x_ref, wt_ref, b_ref, idx_ref, p_ref, pf_ref, if_ref):
    g = pl.program_id(0)
    lt = lax.dot_general(wt_ref[...], x_ref[...], (((0,), (1,)), ((), ())),
                         preferred_element_type=jnp.float32) + b_ref[...]
    m = jnp.max(lt, axis=0, keepdims=True)
    e = jnp.exp(lt - m)
    p_ref[g] = e[1:2, :] / (e[0:1, :] + e[1:2, :])

    @pl.when(g == NB - 1)
    def _select():
        u3 = lax.bitcast_convert_type(p_ref[...], jnp.int32)

        def bit_body(t, pfx):
            cand = pfx | (jnp.int32(1) << (jnp.int32(30) - t))
            c = jnp.sum(jnp.where(u3 >= cand, jnp.int32(1), jnp.int32(0)))
            return jnp.where(c >= K, cand, pfx)
        vstar = lax.fori_loop(0, 31, bit_body, jnp.int32(0))

        utri = (lax.broadcasted_iota(jnp.int32, (RB, RB), 0)
                <= lax.broadcasted_iota(jnp.int32, (RB, RB), 1)
                ).astype(jnp.float32)
        c_col = lax.broadcasted_iota(jnp.int32, (CAP, 1), 0).astype(jnp.float32)
        j_row = lax.broadcasted_iota(jnp.int32, (1, RB), 1).astype(jnp.float32)

        def compact_body(b, carry):
            prow = p_ref[b]
            mrow = (lax.bitcast_convert_type(prow, jnp.int32)
                    >= vstar).astype(jnp.float32)
            cs = jnp.dot(mrow, utri,
                         preferred_element_type=jnp.float32)
            destv = jnp.where(mrow > 0, cs - mrow, -1.0)
            comp = (destv == c_col).astype(jnp.float32)
            fil = jnp.sum(comp, axis=1, keepdims=True)
            pf = jnp.sum(comp * prow, axis=1, keepdims=True)
            jf = jnp.sum(comp * j_row, axis=1, keepdims=True)
            pf_ref[pl.ds(b * CAP, CAP), :] = pf + fil - 1.0
            if_ref[pl.ds(b * CAP, CAP), :] = (
                jf + (b * RB).astype(jnp.float32) * fil)
            return carry
        lax.fori_loop(0, NB, compact_body, jnp.int32(0))

        p_col = pf_ref[...]
        i_col = if_ref[...]
        u_col = lax.bitcast_convert_type(p_col, jnp.int32)
        i_int = i_col.astype(jnp.int32)

        eye = (lax.broadcasted_iota(jnp.int32, (CH, CH), 0)
               == lax.broadcasted_iota(jnp.int32, (CH, CH), 1)
               ).astype(jnp.float32)
        p_rows, i_rows = [], []
        for t in range(M // CH):
            uc = u_col[t * CH:(t + 1) * CH, :]
            ic = i_int[t * CH:(t + 1) * CH, :]
            ub = [((uc >> s) & 255).astype(jnp.float32) for s in (0, 8, 16, 24)]
            ib = [(ic & 255).astype(jnp.float32),
                  (ic >> 8).astype(jnp.float32)]
            ubr = _transpose_planes(ub, eye)
            ibr = _transpose_planes(ib, eye)
            ur = (ubr[0].astype(jnp.int32)
                  | (ubr[1].astype(jnp.int32) << 8)
                  | (ubr[2].astype(jnp.int32) << 16)
                  | (ubr[3].astype(jnp.int32) << 24))
            p_rows.append(lax.bitcast_convert_type(ur, jnp.float32))
            i_rows.append(ibr[0] + ibr[1] * 256.0)
        p_row = jnp.concatenate(p_rows, axis=1)
        i_row = jnp.concatenate(i_rows, axis=1)

        r_row = lax.broadcasted_iota(jnp.int32, (1, K), 1).astype(jnp.float32)
        acc_hi = jnp.zeros((1, K), jnp.float32)
        acc_lo = jnp.zeros((1, K), jnp.float32)
        for t in range(M // CH):
            pd = p_col[t * CH:(t + 1) * CH, :]
            idd = i_col[t * CH:(t + 1) * CH, :]
            ic = i_int[t * CH:(t + 1) * CH, :]
            cmp = (p_row > pd) | ((p_row == pd) & (i_row < idd))
            rank = jnp.sum(cmp.astype(jnp.float32), axis=1, keepdims=True)
            onehot = (rank == r_row).astype(jnp.float32)
            hi = (ic >> 8).astype(jnp.float32)
            lo = (ic & 255).astype(jnp.float32)
            acc_hi = acc_hi + lax.dot_general(
                hi, onehot, (((0,), (0,)), ((), ())),
                preferred_element_type=jnp.float32)
            acc_lo = acc_lo + lax.dot_general(
                lo, onehot, (((0,), (0,)), ((), ())),
                preferred_element_type=jnp.float32)
        idx_ref[...] = (acc_hi * 256.0 + acc_lo).astype(jnp.int32)


def _gather_kernel(idx_ref, x_ref, o_ref, sem):
    def start(j):
        pltpu.make_async_copy(x_ref.at[pl.ds(idx_ref[j], 1)],
                              o_ref.at[pl.ds(j, 1)],
                              sem.at[j % DWIN]).start()

    for j in range(DWIN):
        start(j)

    def body(i, carry):
        pltpu.make_async_copy(x_ref.at[pl.ds(idx_ref[i], 1)],
                              o_ref.at[pl.ds(i, 1)],
                              sem.at[i % DWIN]).wait()

        @pl.when(i + DWIN < K)
        def _():
            start(i + DWIN)
        return carry
    lax.fori_loop(0, K, body, jnp.int32(0))


_SC_NC, _SC_NS = 2, 16
_SC_NW = _SC_NC * _SC_NS
_SC_BW = K // _SC_NW


def _sc_gather_body(idx_hbm, x_hbm, out_hbm, idx_v, rows_v, sem):
    wid = lax.axis_index("s") * _SC_NC + lax.axis_index("c")
    base = wid * _SC_BW
    pltpu.sync_copy(idx_hbm.at[pl.ds(base, _SC_BW)], idx_v)
    pltpu.async_copy(x_hbm.at[idx_v], rows_v, sem).wait()
    pltpu.sync_copy(rows_v, out_hbm.at[pl.ds(base, _SC_BW)])


def _sc_gather(top_idx, x):
    import functools
    from jax.experimental.pallas import tpu_sc as plsc
    mesh = plsc.VectorSubcoreMesh(core_axis_name="c", subcore_axis_name="s")
    return pl.kernel(
        _sc_gather_body,
        out_type=jax.ShapeDtypeStruct((K, D), jnp.float32),
        mesh=mesh,
        scratch_types=[
            pltpu.VMEM((_SC_BW,), jnp.int32),
            pltpu.VMEM((_SC_BW, D), jnp.float32),
            pltpu.SemaphoreType.DMA,
        ],
    )(top_idx, x)


@jax.jit
def kernel(x, W, b):
    top_idx = pl.pallas_call(
        _score_select_kernel,
        grid=(NB,),
        in_specs=[
            pl.BlockSpec((RB, D), lambda g: (g, 0)),
            pl.BlockSpec((D, 2), lambda g: (0, 0)),
            pl.BlockSpec((2, 1), lambda g: (0, 0)),
        ],
        out_specs=pl.BlockSpec((1, K), lambda g: (0, 0)),
        out_shape=jax.ShapeDtypeStruct((1, K), jnp.int32),
        scratch_shapes=[
            pltpu.VMEM((NB, 1, RB), jnp.float32),
            pltpu.VMEM((M, 1), jnp.float32),
            pltpu.VMEM((M, 1), jnp.float32),
        ],
    )(x, W.T, b.reshape(2, 1))

    return _sc_gather(top_idx.reshape(K), x)


# speedup vs baseline: 4.4594x; 1.0492x over previous
import jax
import jax.numpy as jnp
from jax import lax
from jax.experimental import pallas as pl
from jax.experimental.pallas import tpu as pltpu

N = 32768
D = 2048
K = 512
RB = 1024
NB = N // RB
CAP = 64
M = NB * CAP
CH = 512
DWIN = 16


def _transpose_planes(planes, eye):
    return [lax.dot_general(pln, eye, (((0,), (0,)), ((), ())),
                            preferred_element_type=jnp.float32)
            for pln in planes]


def _score_select_kernel(x_ref, wt_ref, b_ref, idx_ref, p_ref, pf_ref, if_ref):
    g = pl.program_id(0)
    lt = lax.dot_general(wt_ref[...], x_ref[...], (((0,), (1,)), ((), ())),
                         preferred_element_type=jnp.float32) + b_ref[...]
    m = jnp.max(lt, axis=0, keepdims=True)
    e = jnp.exp(lt - m)
    p_ref[g] = e[1:2, :] / (e[0:1, :] + e[1:2, :])

    @pl.when(g == NB - 1)
    def _select():
        u3 = lax.bitcast_convert_type(p_ref[...], jnp.int32)

        def bit_body(t, pfx):
            sh = jnp.int32(28) - 2 * t
            c1 = jnp.sum(jnp.where(u3 >= (pfx | (jnp.int32(1) << sh)),
                                   jnp.int32(1), jnp.int32(0)))
            c2 = jnp.sum(jnp.where(u3 >= (pfx | (jnp.int32(2) << sh)),
                                   jnp.int32(1), jnp.int32(0)))
            c3 = jnp.sum(jnp.where(u3 >= (pfx | (jnp.int32(3) << sh)),
                                   jnp.int32(1), jnp.int32(0)))
            m = (jnp.where(c1 >= K, jnp.int32(1), jnp.int32(0))
                 + jnp.where(c2 >= K, jnp.int32(1), jnp.int32(0))
                 + jnp.where(c3 >= K, jnp.int32(1), jnp.int32(0)))
            return pfx | (m << sh)
        vstar = lax.fori_loop(0, 15, bit_body, jnp.int32(0))

        utri = (lax.broadcasted_iota(jnp.int32, (RB, RB), 0)
                <= lax.broadcasted_iota(jnp.int32, (RB, RB), 1)
                ).astype(jnp.float32)
        c_col = lax.broadcasted_iota(jnp.int32, (CAP, 1), 0).astype(jnp.float32)
        j_row = lax.broadcasted_iota(jnp.int32, (1, RB), 1).astype(jnp.float32)

        for b in range(NB):
            prow = p_ref[b]
            mrow = (lax.bitcast_convert_type(prow, jnp.int32)
                    >= vstar).astype(jnp.float32)
            cs = jnp.dot(mrow, utri,
                         preferred_element_type=jnp.float32)
            destv = jnp.where(mrow > 0, cs - mrow, -1.0)
            comp = (destv == c_col).astype(jnp.float32)
            fil = jnp.sum(comp, axis=1, keepdims=True)
            pf = jnp.sum(comp * prow, axis=1, keepdims=True)
            jf = jnp.sum(comp * j_row, axis=1, keepdims=True)
            pf_ref[b * CAP:(b + 1) * CAP, :] = pf + fil - 1.0
            if_ref[b * CAP:(b + 1) * CAP, :] = jf + float(b * RB) * fil

        p_col = pf_ref[...]
        i_col = if_ref[...]
        u_col = lax.bitcast_convert_type(p_col, jnp.int32)
        i_int = i_col.astype(jnp.int32)

        eye = (lax.broadcasted_iota(jnp.int32, (CH, CH), 0)
               == lax.broadcasted_iota(jnp.int32, (CH, CH), 1)
               ).astype(jnp.float32)
        p_rows, i_rows = [], []
        for t in range(M // CH):
            uc = u_col[t * CH:(t + 1) * CH, :]
            ic = i_int[t * CH:(t + 1) * CH, :]
            ub = [((uc >> s) & 255).astype(jnp.float32) for s in (0, 8, 16, 24)]
            ib = [(ic & 255).astype(jnp.float32),
                  (ic >> 8).astype(jnp.float32)]
            ubr = _transpose_planes(ub, eye)
            ibr = _transpose_planes(ib, eye)
            ur = (ubr[0].astype(jnp.int32)
                  | (ubr[1].astype(jnp.int32) << 8)
                  | (ubr[2].astype(jnp.int32) << 16)
                  | (ubr[3].astype(jnp.int32) << 24))
            p_rows.append(lax.bitcast_convert_type(ur, jnp.float32))
            i_rows.append(ibr[0] + ibr[1] * 256.0)
        p_row = jnp.concatenate(p_rows, axis=1)
        i_row = jnp.concatenate(i_rows, axis=1)

        r_row = lax.broadcasted_iota(jnp.int32, (1, K), 1).astype(jnp.float32)
        acc_hi = jnp.zeros((1, K), jnp.float32)
        acc_lo = jnp.zeros((1, K), jnp.float32)
        for t in range(M // CH):
            pd = p_col[t * CH:(t + 1) * CH, :]
            idd = i_col[t * CH:(t + 1) * CH, :]
            ic = i_int[t * CH:(t + 1) * CH, :]
            cmp = (p_row > pd) | ((p_row == pd) & (i_row < idd))
            rank = jnp.sum(cmp.astype(jnp.float32), axis=1, keepdims=True)
            onehot = (rank == r_row).astype(jnp.float32)
            hi = (ic >> 8).astype(jnp.float32)
            lo = (ic & 255).astype(jnp.float32)
            acc_hi = acc_hi + lax.dot_general(
                hi, onehot, (((0,), (0,)), ((), ())),
                preferred_element_type=jnp.float32)
            acc_lo = acc_lo + lax.dot_general(
                lo, onehot, (((0,), (0,)), ((), ())),
                preferred_element_type=jnp.float32)
        idx_ref[...] = (acc_hi * 256.0 + acc_lo).astype(jnp.int32)


def _gather_kernel(idx_ref, x_ref, o_ref, sem):
    def start(j):
        pltpu.make_async_copy(x_ref.at[pl.ds(idx_ref[j], 1)],
                              o_ref.at[pl.ds(j, 1)],
                              sem.at[j % DWIN]).start()

    for j in range(DWIN):
        start(j)

    def body(i, carry):
        pltpu.make_async_copy(x_ref.at[pl.ds(idx_ref[i], 1)],
                              o_ref.at[pl.ds(i, 1)],
                              sem.at[i % DWIN]).wait()

        @pl.when(i + DWIN < K)
        def _():
            start(i + DWIN)
        return carry
    lax.fori_loop(0, K, body, jnp.int32(0))


_SC_NC, _SC_NS = 2, 16
_SC_NW = _SC_NC * _SC_NS
_SC_BW = K // _SC_NW


def _sc_gather_body(idx_hbm, x_hbm, out_hbm, idx_v, rows_v, sem):
    wid = lax.axis_index("s") * _SC_NC + lax.axis_index("c")
    base = wid * _SC_BW
    pltpu.sync_copy(idx_hbm.at[pl.ds(base, _SC_BW)], idx_v)
    pltpu.async_copy(x_hbm.at[idx_v], rows_v, sem).wait()
    pltpu.sync_copy(rows_v, out_hbm.at[pl.ds(base, _SC_BW)])


def _sc_gather(top_idx, x):
    import functools
    from jax.experimental.pallas import tpu_sc as plsc
    mesh = plsc.VectorSubcoreMesh(core_axis_name="c", subcore_axis_name="s")
    return pl.kernel(
        _sc_gather_body,
        out_type=jax.ShapeDtypeStruct((K, D), jnp.float32),
        mesh=mesh,
        scratch_types=[
            pltpu.VMEM((_SC_BW,), jnp.int32),
            pltpu.VMEM((_SC_BW, D), jnp.float32),
            pltpu.SemaphoreType.DMA,
        ],
    )(top_idx, x)


@jax.jit
def kernel(x, W, b):
    top_idx = pl.pallas_call(
        _score_select_kernel,
        grid=(NB,),
        in_specs=[
            pl.BlockSpec((RB, D), lambda g: (g, 0)),
            pl.BlockSpec((D, 2), lambda g: (0, 0)),
            pl.BlockSpec((2, 1), lambda g: (0, 0)),
        ],
        out_specs=pl.BlockSpec((1, K), lambda g: (0, 0)),
        out_shape=jax.ShapeDtypeStruct((1, K), jnp.int32),
        scratch_shapes=[
            pltpu.VMEM((NB, 1, RB), jnp.float32),
            pltpu.VMEM((M, 1), jnp.float32),
            pltpu.VMEM((M, 1), jnp.float32),
        ],
    )(x, W.T, b.reshape(2, 1))

    return _sc_gather(top_idx.reshape(K), x)
